# Initial kernel scaffold; baseline (speedup 1.0000x reference)
import jax
import jax.numpy as jnp
from jax.experimental import pallas as pl


def kernel(reference, attr, batch_index):
    raise NotImplementedError("write your pallas kernel here")


# speedup vs baseline: 3.9009x; 3.9009x over previous
import functools

import jax
import jax.numpy as jnp
from jax import lax
from jax.experimental import pallas as pl
from jax.experimental.pallas import tpu as pltpu
from jax.experimental.pallas import tpu_sc as plsc


def _segment_sum_sc(attr, idx, num_segments):
  N, F = attr.shape
  info = plsc.get_sparse_core_info()
  NC, NS, L = info.num_cores, info.num_subcores, info.num_lanes
  NW = NC * NS
  rows_per_w = N // NW
  R = 80
  steps = rows_per_w // R

  mesh = plsc.VectorSubcoreMesh(core_axis_name="c", subcore_axis_name="s")

  @functools.partial(
      pl.kernel,
      mesh=mesh,
      out_type=jax.ShapeDtypeStruct((NC, num_segments, F), jnp.float32),
      scratch_types=[
          pltpu.VMEM((R,), jnp.int32),
          pltpu.VMEM((R, F), jnp.float32),
          pltpu.VMEM((num_segments, F), jnp.float32),
          pltpu.VMEM_SHARED((num_segments, F), jnp.float32),
      ],
  )
  def k(attr_hbm, idx_hbm, out_hbm, idx_v, rows_v, zero_v, acc_sh):
    cid = lax.axis_index("c")
    sid = lax.axis_index("s")

    @pl.when(sid == 0)
    def _():
      def zrow(i, carry):
        for j in range(F // L):
          zero_v[i, pl.ds(j * L, L)] = jnp.zeros((L,), jnp.float32)
        return carry
      lax.fori_loop(0, num_segments, zrow, 0)
      pltpu.sync_copy(zero_v, acc_sh)

    plsc.subcore_barrier()

    wid = sid * NC + cid
    base = wid * rows_per_w

    def body(t, carry):
      r0 = base + t * R
      pltpu.sync_copy(idx_hbm.at[pl.ds(r0, R)], idx_v)
      pltpu.sync_copy(attr_hbm.at[pl.ds(r0, R)], rows_v)
      pltpu.sync_copy(rows_v, acc_sh.at[idx_v], add=True)
      return carry

    lax.fori_loop(0, steps, body, 0)

    plsc.subcore_barrier()

    @pl.when(sid == 0)
    def _():
      pltpu.sync_copy(acc_sh, out_hbm.at[cid])

  return k(attr, idx)


def kernel(reference, attr, batch_index):
  num_segments = reference.shape[0]
  idx = batch_index.astype(jnp.int32)
  partials = _segment_sum_sc(attr, idx, num_segments)
  return partials[0] + partials[1]


# speedup vs baseline: 5.7421x; 1.4720x over previous
import functools

import jax
import jax.numpy as jnp
from jax import lax
from jax.experimental import pallas as pl
from jax.experimental.pallas import tpu as pltpu
from jax.experimental.pallas import tpu_sc as plsc


def _segment_sum_sc(attr, idx, num_segments):
  N, F = attr.shape
  info = plsc.get_sparse_core_info()
  NC, NS, L = info.num_cores, info.num_subcores, info.num_lanes
  NW = NC * NS
  rows_per_w = N // NW
  R = 80
  steps = rows_per_w // R
  NBUF = 5
  groups = steps // NBUF

  idx3 = idx.reshape(NW, steps, R)

  mesh = plsc.VectorSubcoreMesh(core_axis_name="c", subcore_axis_name="s")

  @functools.partial(
      pl.kernel,
      mesh=mesh,
      out_type=jax.ShapeDtypeStruct((NC, num_segments, F), jnp.float32),
      scratch_types=[
          pltpu.VMEM((steps, R), jnp.int32),
          [pltpu.VMEM((R, F), jnp.float32) for _ in range(NBUF)],
          pltpu.VMEM((num_segments, F), jnp.float32),
          pltpu.VMEM_SHARED((num_segments, F), jnp.float32),
          pltpu.SemaphoreType.DMA((NBUF,)),
          pltpu.SemaphoreType.DMA,
      ],
  )
  def k(attr_hbm, idx_hbm, out_hbm, idx_all, rows, zero_v, acc_sh, gsem, ssem):
    cid = lax.axis_index("c")
    sid = lax.axis_index("s")

    @pl.when(sid == 0)
    def _():
      def zrow(i, carry):
        for j in range(F // L):
          zero_v[i, pl.ds(j * L, L)] = jnp.zeros((L,), jnp.float32)
        return carry
      lax.fori_loop(0, num_segments, zrow, 0)
      pltpu.sync_copy(zero_v, acc_sh)

    wid = sid * NC + cid
    base = wid * rows_per_w

    pltpu.sync_copy(idx_hbm.at[wid], idx_all)

    plsc.subcore_barrier()

    def gather(t, b):
      return pltpu.async_copy(
          attr_hbm.at[pl.ds(base + t * R, R)], rows[b], gsem.at[b])

    for b in range(NBUF):
      gather(b, b)

    def body(g, carry):
      for b in range(NBUF):
        t = g * NBUF + b
        pltpu.make_async_copy(
            attr_hbm.at[pl.ds(base + t * R, R)], rows[b], gsem.at[b]).wait()
        sc = pltpu.async_copy(rows[b], acc_sh.at[idx_all.at[t]], ssem,
                              add=True)
        sc.wait()
        @pl.when(t + NBUF < steps)
        def _():
          gather(t + NBUF, b)
      return carry

    lax.fori_loop(0, groups, body, 0)

    plsc.subcore_barrier()

    @pl.when(sid == 0)
    def _():
      pltpu.sync_copy(acc_sh, out_hbm.at[cid])

  return k(attr, idx3)


def kernel(reference, attr, batch_index):
  num_segments = reference.shape[0]
  idx = batch_index.astype(jnp.int32)
  partials = _segment_sum_sc(attr, idx, num_segments)
  return partials[0] + partials[1]
